# baseline (speedup 1.0000x reference)
import functools
import math

import jax
import jax.numpy as jnp
from jax import lax
from jax.experimental import pallas as pl
from jax.experimental.pallas import tpu as pltpu
from jax.experimental.pallas import tpu_sc as plsc

MAX_POS = 32768
HEAD_DIM = 128
HALF = HEAD_DIM // 2
CACHE_DIM = 2 * HEAD_DIM

NUM_CORES = 2
NUM_SUBCORES = 16
NW = NUM_CORES * NUM_SUBCORES

BATCH = 4
SEQ = 8192
W_PER_B = NW // BATCH
PER_W = SEQ // W_PER_B
CHUNK = 128
NCH = PER_W // CHUNK
NBUF = 4
LOOK = 2


def _sin_gather_body(cache, idx, sin_out, idx_raw, rows, gsem, wsem):
    wid = lax.axis_index("s") * NUM_CORES + lax.axis_index("c")
    bi = wid // W_PER_B
    col = (wid % W_PER_B) * PER_W

    pltpu.sync_copy(idx.at[bi, pl.ds(col, PER_W)], idx_raw)

    def fire_gather(c, b):
        sl = idx_raw.at[pl.ds(c * CHUNK, CHUNK)]
        pltpu.async_copy(
            cache.at[sl, pl.ds(HEAD_DIM, HEAD_DIM)], rows.at[b], gsem.at[b]
        )

    def fire_write(c, b):
        pltpu.async_copy(
            rows.at[b], sin_out.at[bi, pl.ds(col + c * CHUNK, CHUNK)], wsem.at[b]
        )

    def drain_write(c, b):
        pltpu.make_async_copy(
            rows.at[b], sin_out.at[bi, pl.ds(col + c * CHUNK, CHUNK)], wsem.at[b]
        ).wait()

    def drain_gather(b):
        pltpu.make_async_copy(
            cache.at[pl.ds(0, CHUNK), pl.ds(HEAD_DIM, HEAD_DIM)],
            rows.at[b],
            gsem.at[b],
        ).wait()

    for c in range(LOOK):
        fire_gather(c, c)

    def step(c, carry):
        b = c % NBUF

        @pl.when(c + LOOK < NCH)
        def _fire_next():
            nb = (c + LOOK) % NBUF
            @pl.when(c + LOOK >= NBUF)
            def _drain():
                drain_write(c + LOOK - NBUF, nb)

            fire_gather(c + LOOK, nb)

        drain_gather(b)
        fire_write(c, b)
        return carry

    lax.fori_loop(0, NCH, step, 0)

    for c in range(NCH - NBUF, NCH):
        drain_write(c, c % NBUF)


def _sin_gather(cache, idx):
    mesh = plsc.VectorSubcoreMesh(core_axis_name="c", subcore_axis_name="s")
    out_ty = jax.ShapeDtypeStruct((BATCH, SEQ, HEAD_DIM), jnp.float32)
    run = pl.kernel(
        _sin_gather_body,
        out_type=out_ty,
        mesh=mesh,
        scratch_types=[
            pltpu.VMEM((PER_W,), jnp.int32),
            pltpu.VMEM((NBUF, CHUNK, HEAD_DIM), jnp.float32),
            pltpu.SemaphoreType.DMA((NBUF,)),
            pltpu.SemaphoreType.DMA((NBUF,)),
        ],
    )
    return run(cache, idx)


SEQ_BLK = 512
LN_BASE = math.log(10000.0)


def _cos_compute_body(pos_ref, cos_ref):
    l = lax.broadcasted_iota(jnp.int32, (1, 1, HEAD_DIM), 2)
    m = jnp.where(l < HALF, l, l - HALF).astype(jnp.float32)
    inv_freq = jnp.exp(m * (-LN_BASE / HALF))
    p = pos_ref[...].astype(jnp.float32)
    emb = p[:, :, None] * inv_freq
    cos_ref[...] = jnp.cos(emb)


def _cos_compute(idx):
    grid = (SEQ // SEQ_BLK,)
    return pl.pallas_call(
        _cos_compute_body,
        grid=grid,
        in_specs=[pl.BlockSpec((BATCH, SEQ_BLK), lambda i: (0, i))],
        out_specs=pl.BlockSpec((BATCH, SEQ_BLK, HEAD_DIM), lambda i: (0, i, 0)),
        out_shape=jax.ShapeDtypeStruct((BATCH, SEQ, HEAD_DIM), jnp.float32),
    )(idx)


@jax.jit
def _rotary(cache, idx):
    return _cos_compute(idx), _sin_gather(cache, idx)


def kernel(x, position_ids, cos_sin_cache):
    del x
    cos, sin = _rotary(cos_sin_cache, position_ids)
    return (cos, sin)


# speedup vs baseline: 1.5212x; 1.5212x over previous
import functools

import jax
import jax.numpy as jnp
from jax import lax
from jax.experimental import pallas as pl
from jax.experimental.pallas import tpu as pltpu
from jax.experimental.pallas import tpu_sc as plsc

MAX_POS = 32768
HEAD_DIM = 128
CACHE_DIM = 2 * HEAD_DIM

NUM_CORES = 2
NUM_SUBCORES = 16
NW = NUM_CORES * NUM_SUBCORES

BATCH = 4
SEQ = 8192
W_PER_B = NW // BATCH
PER_W = SEQ // W_PER_B
CHUNK = 128
NCH = PER_W // CHUNK
NBUF = 3
LOOK = 2


def _rotary_gather_body(cache, idx, cos_out, sin_out, idx_raw, cbuf, sbuf,
                        gsem, wsem):
    wid = lax.axis_index("s") * NUM_CORES + lax.axis_index("c")
    bi = wid // W_PER_B
    col = (wid % W_PER_B) * PER_W

    pltpu.sync_copy(idx.at[bi, pl.ds(col, PER_W)], idx_raw)

    def fire_gather(c, b):
        sl = idx_raw.at[pl.ds(c * CHUNK, CHUNK)]
        pltpu.async_copy(cache.at[sl, pl.ds(0, HEAD_DIM)], cbuf.at[b],
                         gsem.at[b])
        pltpu.async_copy(cache.at[sl, pl.ds(HEAD_DIM, HEAD_DIM)], sbuf.at[b],
                         gsem.at[b])

    def fire_writes(c, b):
        dst = pl.ds(col + c * CHUNK, CHUNK)
        pltpu.async_copy(cbuf.at[b], cos_out.at[bi, dst], wsem.at[b])
        pltpu.async_copy(sbuf.at[b], sin_out.at[bi, dst], wsem.at[b])

    def drain_writes(c, b):
        dst = pl.ds(col + c * CHUNK, CHUNK)
        pltpu.make_async_copy(cbuf.at[b], cos_out.at[bi, dst],
                              wsem.at[b]).wait()
        pltpu.make_async_copy(sbuf.at[b], sin_out.at[bi, dst],
                              wsem.at[b]).wait()

    def drain_gather(b):
        pltpu.make_async_copy(cache.at[pl.ds(0, CHUNK), pl.ds(0, HEAD_DIM)],
                              cbuf.at[b], gsem.at[b]).wait()
        pltpu.make_async_copy(cache.at[pl.ds(0, CHUNK), pl.ds(0, HEAD_DIM)],
                              sbuf.at[b], gsem.at[b]).wait()

    for c in range(LOOK):
        fire_gather(c, c)

    def step(c, carry):
        b = c % NBUF

        @pl.when(c + LOOK < NCH)
        def _fire_next():
            nb = (c + LOOK) % NBUF
            @pl.when(c + LOOK >= NBUF)
            def _drain():
                drain_writes(c + LOOK - NBUF, nb)

            fire_gather(c + LOOK, nb)

        drain_gather(b)
        fire_writes(c, b)
        return carry

    lax.fori_loop(0, NCH, step, 0)

    for c in range(NCH - NBUF, NCH):
        drain_writes(c, c % NBUF)


@jax.jit
def _rotary_gather(cache, idx):
    mesh = plsc.VectorSubcoreMesh(core_axis_name="c", subcore_axis_name="s")
    out_ty = jax.ShapeDtypeStruct((BATCH, SEQ, HEAD_DIM), jnp.float32)
    run = pl.kernel(
        _rotary_gather_body,
        out_type=(out_ty, out_ty),
        mesh=mesh,
        scratch_types=[
            pltpu.VMEM((PER_W,), jnp.int32),
            pltpu.VMEM((NBUF, CHUNK, HEAD_DIM), jnp.float32),
            pltpu.VMEM((NBUF, CHUNK, HEAD_DIM), jnp.float32),
            pltpu.SemaphoreType.DMA((NBUF,)),
            pltpu.SemaphoreType.DMA((NBUF,)),
        ],
    )
    return run(cache, idx)


def kernel(x, position_ids, cos_sin_cache):
    del x
    return _rotary_gather(cos_sin_cache, position_ids)


# speedup vs baseline: 1.6041x; 1.0545x over previous
import functools

import jax
import jax.numpy as jnp
from jax import lax
from jax.experimental import pallas as pl
from jax.experimental.pallas import tpu as pltpu
from jax.experimental.pallas import tpu_sc as plsc

MAX_POS = 32768
HEAD_DIM = 128
CACHE_DIM = 2 * HEAD_DIM

NUM_CORES = 2
NUM_SUBCORES = 16
NW = NUM_CORES * NUM_SUBCORES

BATCH = 4
SEQ = 8192
W_PER_B = NW // BATCH
PER_W = SEQ // W_PER_B
CHUNK = 128
NCH = PER_W // CHUNK
NBUF = 3
LOOK = 1


def _rotary_gather_body(cache, idx, cos_out, sin_out, idx_raw, rows, gsem, wsem):
    wid = lax.axis_index("s") * NUM_CORES + lax.axis_index("c")
    bi = wid // W_PER_B
    col = (wid % W_PER_B) * PER_W

    pltpu.sync_copy(idx.at[bi, pl.ds(col, PER_W)], idx_raw)

    def fire_gather(c, b):
        pltpu.async_copy(
            cache.at[idx_raw.at[pl.ds(c * CHUNK, CHUNK)]], rows.at[b],
            gsem.at[b]
        )

    def fire_writes(c, b):
        dst = pl.ds(col + c * CHUNK, CHUNK)
        pltpu.async_copy(rows.at[b, :, pl.ds(0, HEAD_DIM)],
                         cos_out.at[bi, dst], wsem.at[b])
        pltpu.async_copy(rows.at[b, :, pl.ds(HEAD_DIM, HEAD_DIM)],
                         sin_out.at[bi, dst], wsem.at[b])

    def drain_writes(c, b):
        dst = pl.ds(col + c * CHUNK, CHUNK)
        pltpu.make_async_copy(rows.at[b, :, pl.ds(0, HEAD_DIM)],
                              cos_out.at[bi, dst], wsem.at[b]).wait()
        pltpu.make_async_copy(rows.at[b, :, pl.ds(HEAD_DIM, HEAD_DIM)],
                              sin_out.at[bi, dst], wsem.at[b]).wait()

    def drain_gather(b):
        pltpu.make_async_copy(cache.at[pl.ds(0, CHUNK)], rows.at[b],
                              gsem.at[b]).wait()

    for c in range(LOOK):
        fire_gather(c, c)

    def step(c, carry):
        b = c % NBUF

        @pl.when(c + LOOK < NCH)
        def _fire_next():
            nb = (c + LOOK) % NBUF
            @pl.when(c + LOOK >= NBUF)
            def _drain():
                drain_writes(c + LOOK - NBUF, nb)

            fire_gather(c + LOOK, nb)

        drain_gather(b)
        fire_writes(c, b)
        return carry

    lax.fori_loop(0, NCH, step, 0)

    for c in range(NCH - NBUF, NCH):
        drain_writes(c, c % NBUF)


@jax.jit
def _rotary_gather(cache, idx):
    mesh = plsc.VectorSubcoreMesh(core_axis_name="c", subcore_axis_name="s")
    out_ty = jax.ShapeDtypeStruct((BATCH, SEQ, HEAD_DIM), jnp.float32)
    run = pl.kernel(
        _rotary_gather_body,
        out_type=(out_ty, out_ty),
        mesh=mesh,
        scratch_types=[
            pltpu.VMEM((PER_W,), jnp.int32),
            pltpu.VMEM((NBUF, CHUNK, CACHE_DIM), jnp.float32),
            pltpu.SemaphoreType.DMA((NBUF,)),
            pltpu.SemaphoreType.DMA((NBUF,)),
        ],
    )
    return run(cache, idx)


def kernel(x, position_ids, cos_sin_cache):
    del x
    return _rotary_gather(cos_sin_cache, position_ids)


# speedup vs baseline: 1.6245x; 1.0127x over previous
import functools

import jax
import jax.numpy as jnp
from jax import lax
from jax.experimental import pallas as pl
from jax.experimental.pallas import tpu as pltpu
from jax.experimental.pallas import tpu_sc as plsc

MAX_POS = 32768
HEAD_DIM = 128
CACHE_DIM = 2 * HEAD_DIM

NUM_CORES = 2
NUM_SUBCORES = 16
NW = NUM_CORES * NUM_SUBCORES

BATCH = 4
SEQ = 8192
W_PER_B = NW // BATCH
PER_W = SEQ // W_PER_B
CHUNK = 128
NCH = PER_W // CHUNK
NBUF = 3
LOOK = 2


def _rotary_gather_body(cache, idx, cos_out, sin_out, idx_raw, rows, gsem, wsem):
    wid = lax.axis_index("s") * NUM_CORES + lax.axis_index("c")
    bi = wid // W_PER_B
    col = (wid % W_PER_B) * PER_W

    pltpu.sync_copy(idx.at[bi, pl.ds(col, PER_W)], idx_raw)

    def fire_gather(c, b):
        pltpu.async_copy(
            cache.at[idx_raw.at[pl.ds(c * CHUNK, CHUNK)]], rows.at[b],
            gsem.at[b]
        )

    def fire_writes(c, b):
        dst = pl.ds(col + c * CHUNK, CHUNK)
        pltpu.async_copy(rows.at[b, :, pl.ds(0, HEAD_DIM)],
                         cos_out.at[bi, dst], wsem.at[b])
        pltpu.async_copy(rows.at[b, :, pl.ds(HEAD_DIM, HEAD_DIM)],
                         sin_out.at[bi, dst], wsem.at[b])

    def drain_writes(c, b):
        dst = pl.ds(col + c * CHUNK, CHUNK)
        pltpu.make_async_copy(rows.at[b, :, pl.ds(0, HEAD_DIM)],
                              cos_out.at[bi, dst], wsem.at[b]).wait()
        pltpu.make_async_copy(rows.at[b, :, pl.ds(HEAD_DIM, HEAD_DIM)],
                              sin_out.at[bi, dst], wsem.at[b]).wait()

    def drain_gather(b):
        pltpu.make_async_copy(cache.at[pl.ds(0, CHUNK)], rows.at[b],
                              gsem.at[b]).wait()

    for c in range(LOOK):
        fire_gather(c, c)

    def step(c, carry):
        b = c % NBUF

        @pl.when(c + LOOK < NCH)
        def _fire_next():
            nb = (c + LOOK) % NBUF
            @pl.when(c + LOOK >= NBUF)
            def _drain():
                drain_writes(c + LOOK - NBUF, nb)

            fire_gather(c + LOOK, nb)

        drain_gather(b)
        fire_writes(c, b)
        return carry

    lax.fori_loop(0, NCH, step, 0)

    for c in range(NCH - NBUF, NCH):
        drain_writes(c, c % NBUF)


@jax.jit
def _rotary_gather(cache, idx):
    mesh = plsc.VectorSubcoreMesh(core_axis_name="c", subcore_axis_name="s")
    out_ty = jax.ShapeDtypeStruct((BATCH, SEQ, HEAD_DIM), jnp.float32)
    run = pl.kernel(
        _rotary_gather_body,
        out_type=(out_ty, out_ty),
        mesh=mesh,
        scratch_types=[
            pltpu.VMEM((PER_W,), jnp.int32),
            pltpu.VMEM((NBUF, CHUNK, CACHE_DIM), jnp.float32),
            pltpu.SemaphoreType.DMA((NBUF,)),
            pltpu.SemaphoreType.DMA((NBUF,)),
        ],
    )
    return run(cache, idx)


def kernel(x, position_ids, cos_sin_cache):
    del x
    return _rotary_gather(cos_sin_cache, position_ids)
